# baseline (speedup 1.0000x reference)
import jax
import jax.numpy as jnp
from jax.experimental import pallas as pl

NUM_NODES = 207
RNN_UNITS = 64
K_DIFF = 2
NUM_CLASSES = 5
INPUT_DIM = 2
BATCH = 32
SEQ_LEN = 12
_PREC = jax.lax.Precision.DEFAULT
NM = K_DIFF + 1
NP = BATCH // 2
NN = 208
HL = 2 * RNN_UNITS
GL = 2 * HL


def _split_xh(W, in_dim):
    out = W.shape[1]
    W3 = jnp.transpose(W.reshape(in_dim + RNN_UNITS, NM, out), (1, 0, 2))
    return W3[:, :in_dim, :], W3[:, in_dim:, :]


def _dup_gate(Wb):
    f = Wb.shape[0]
    W3 = Wb.reshape(f, 2, RNN_UNITS)
    eye = jnp.eye(2, dtype=Wb.dtype)
    return jnp.einsum('fgu,ab->afgbu', W3, eye).reshape(2 * f, GL)


def _dup_cand(Wb):
    f = Wb.shape[0]
    eye = jnp.eye(2, dtype=Wb.dtype)
    return jnp.einsum('fu,ab->afbu', Wb, eye).reshape(2 * f, HL)


def _cheb(S, x3):
    y1 = jax.lax.dot_general(
        S, x3, (((1,), (0,)), ((), ())),
        preferred_element_type=jnp.float32,
        precision=_PREC).astype(jnp.bfloat16)
    y2 = (2.0 * jax.lax.dot_general(
        S, y1, (((1,), (0,)), ((), ())),
        preferred_element_type=jnp.float32, precision=_PREC)
          - x3.astype(jnp.float32)).astype(jnp.bfloat16)
    return x3, y1, y2


def _dot3(a3, w):
    return jax.lax.dot_general(
        a3, w, (((2,), (0,)), ((), ())),
        preferred_element_type=jnp.float32, precision=_PREC)


def _cell(S, pair, h, W3k, Wr, bg, bc):
    q0, q1, q2 = _cheb(S, pair)
    acc = _dot3(q0, W3k[0]) + _dot3(q1, W3k[1]) + _dot3(q2, W3k[2])
    gates = 0.5 * jnp.tanh(0.5 * (acc[..., :GL] + bg)) + 0.5
    r = gates[..., :HL]
    u = gates[..., HL:]
    r0, r1, r2 = _cheb(S, (r * h).astype(jnp.bfloat16))
    c = jnp.tanh(acc[..., GL:] + _dot3(r0, Wr[0]) + _dot3(r1, Wr[1])
                 + _dot3(r2, Wr[2]) + bc)
    return u * h + (1.0 - u) * c


def _body(x_ref, s_ref, w0_ref, wr0_ref, bg0_ref, bc0_ref,
          w1_ref, wr1_ref, bg1_ref, bc1_ref, wfc_ref, bfc_ref,
          mask_ref, out_ref):
    S = s_ref[...]
    w0 = [w0_ref[k] for k in range(NM)]
    wr0 = [wr0_ref[k] for k in range(NM)]
    w1 = [w1_ref[k] for k in range(NM)]
    wr1 = [wr1_ref[k] for k in range(NM)]
    bg0 = bg0_ref[...]
    bc0 = bc0_ref[...]
    bg1 = bg1_ref[...]
    bc1 = bc1_ref[...]

    def step(t, carry):
        h0, h1, last = carry
        xt = x_ref[t].T.reshape(NN, NP, 2 * INPUT_DIM)
        pair0 = jnp.concatenate([h0, xt], axis=2).astype(jnp.bfloat16)
        h0 = _cell(S, pair0, h0, w0, wr0, bg0, bc0)
        pair1 = jnp.concatenate([h0, h1], axis=2).astype(jnp.bfloat16)
        h1 = _cell(S, pair1, h1, w1, wr1, bg1, bc1)
        last = last + h1 * mask_ref[t].reshape(1, NP, HL)
        return h0, h1, last

    h0 = jnp.zeros((NN, NP, HL), jnp.float32)
    h1 = jnp.zeros((NN, NP, HL), jnp.float32)
    last = jnp.zeros((NN, NP, HL), jnp.float32)
    h0, h1, last = jax.lax.fori_loop(0, SEQ_LEN, step, (h0, h1, last))

    logits = _dot3(jax.nn.relu(last), wfc_ref[...]) + bfc_ref[...]
    out_ref[...] = jnp.max(logits[:NUM_NODES], axis=0)


def kernel(input_seq, seq_lengths, supports, Wg0, bg0, Wc0, bc0,
           Wg1, bg1, Wc1, bc1, W_fc, b_fc):
    xseq = jnp.transpose(
        input_seq.reshape(NP, 2, SEQ_LEN, NUM_NODES, INPUT_DIM),
        (2, 1, 4, 3, 0))
    xseq = jnp.pad(xseq, ((0, 0), (0, 0), (0, 0), (0, NN - NUM_NODES),
                          (0, 0))).reshape(SEQ_LEN, 2 * INPUT_DIM, NN * NP)
    S = jnp.pad(supports[0], ((0, NN - NUM_NODES), (0, NN - NUM_NODES)))

    wg0x, wg0h = _split_xh(Wg0, INPUT_DIM)
    wc0x, wc0h = _split_xh(Wc0, INPUT_DIM)
    wg1x, wg1h = _split_xh(Wg1, RNN_UNITS)
    wc1x, wc1h = _split_xh(Wc1, RNN_UNITS)

    w0, wr0, w1, wr1 = [], [], [], []
    for k in range(NM):
        top = jnp.concatenate(
            [_dup_gate(wg0h[k]), jnp.zeros((HL, HL), jnp.float32)], axis=1)
        bot = jnp.concatenate(
            [_dup_gate(wg0x[k]), _dup_cand(wc0x[k])], axis=1)
        w0.append(jnp.concatenate([top, bot], axis=0))
        wr0.append(_dup_cand(wc0h[k]))
        top = jnp.concatenate(
            [_dup_gate(wg1x[k]), _dup_cand(wc1x[k])], axis=1)
        bot = jnp.concatenate(
            [_dup_gate(wg1h[k]), jnp.zeros((HL, HL), jnp.float32)], axis=1)
        w1.append(jnp.concatenate([top, bot], axis=0))
        wr1.append(_dup_cand(wc1h[k]))
    w0 = jnp.stack(w0)
    wr0 = jnp.stack(wr0)
    w1 = jnp.stack(w1)
    wr1 = jnp.stack(wr1)

    def gate_bias(b):
        return jnp.broadcast_to(b.reshape(2, 1, RNN_UNITS),
                                (2, 2, RNN_UNITS)).reshape(1, GL)

    def cand_bias(b):
        return jnp.broadcast_to(b.reshape(1, RNN_UNITS),
                                (2, RNN_UNITS)).reshape(1, HL)

    eye = jnp.eye(2, dtype=W_fc.dtype)
    wfc2 = jnp.einsum('uc,ab->aubc', W_fc, eye).reshape(HL, 2 * NUM_CLASSES)
    bfc2 = jnp.broadcast_to(b_fc.reshape(1, NUM_CLASSES),
                            (2, NUM_CLASSES)).reshape(1, 2 * NUM_CLASSES)

    idx = jnp.clip(seq_lengths - 1, 0, SEQ_LEN - 1)
    onehot = (jnp.arange(SEQ_LEN)[:, None] == idx[None, :]).astype(jnp.float32)
    mask = jnp.repeat(onehot.reshape(SEQ_LEN, NP, 2, 1), RNN_UNITS,
                      axis=3).reshape(SEQ_LEN, NP, HL)

    bf = jnp.bfloat16
    pooled2 = pl.pallas_call(
        _body,
        out_shape=jax.ShapeDtypeStruct((NP, 2 * NUM_CLASSES), jnp.float32),
    )(xseq, S.astype(bf), w0.astype(bf), wr0.astype(bf),
      gate_bias(bg0), cand_bias(bc0), w1.astype(bf), wr1.astype(bf),
      gate_bias(bg1), cand_bias(bc1), wfc2.astype(bf), bfc2, mask)
    return pooled2.reshape(BATCH, NUM_CLASSES)


# speedup vs baseline: 1.4766x; 1.4766x over previous
import jax
import jax.numpy as jnp
from jax.experimental import pallas as pl

NUM_NODES = 207
RNN_UNITS = 64
K_DIFF = 2
NUM_CLASSES = 5
INPUT_DIM = 2
BATCH = 32
SEQ_LEN = 12
_PREC = jax.lax.Precision.DEFAULT
NM = K_DIFF + 1
NP = BATCH // 2
NN = 208
HL = 2 * RNN_UNITS
GL = 2 * HL


def _split_xh(W, in_dim):
    out = W.shape[1]
    W3 = jnp.transpose(W.reshape(in_dim + RNN_UNITS, NM, out), (1, 0, 2))
    return W3[:, :in_dim, :], W3[:, in_dim:, :]


def _dup_gate(Wb):
    f = Wb.shape[0]
    W3 = Wb.reshape(f, 2, RNN_UNITS)
    eye = jnp.eye(2, dtype=Wb.dtype)
    return jnp.einsum('fgu,ab->afgbu', W3, eye).reshape(2 * f, GL)


def _dup_cand(Wb):
    f = Wb.shape[0]
    eye = jnp.eye(2, dtype=Wb.dtype)
    return jnp.einsum('fu,ab->afbu', Wb, eye).reshape(2 * f, HL)


def _cheb(S, x3):
    y1 = jax.lax.dot_general(
        S, x3, (((2,), (1,)), ((0,), (0,))),
        preferred_element_type=jnp.float32,
        precision=_PREC).astype(jnp.bfloat16)
    y2 = (2.0 * jax.lax.dot_general(
        S, y1, (((2,), (1,)), ((0,), (0,))),
        preferred_element_type=jnp.float32, precision=_PREC)
          - x3.astype(jnp.float32)).astype(jnp.bfloat16)
    return x3, y1, y2


def _dot3(a3, w):
    return jax.lax.dot_general(
        a3, w, (((2,), (0,)), ((), ())),
        preferred_element_type=jnp.float32, precision=_PREC)


def _cell(S, pair, h, W3k, Wr, bg, bc):
    q0, q1, q2 = _cheb(S, pair)
    acc = _dot3(q0, W3k[0]) + _dot3(q1, W3k[1]) + _dot3(q2, W3k[2])
    gates = 0.5 * jnp.tanh(0.5 * (acc[..., :GL] + bg)) + 0.5
    r = gates[..., :HL]
    u = gates[..., HL:]
    r0, r1, r2 = _cheb(S, (r * h).astype(jnp.bfloat16))
    c = jnp.tanh(acc[..., GL:] + _dot3(r0, Wr[0]) + _dot3(r1, Wr[1])
                 + _dot3(r2, Wr[2]) + bc)
    return u * h + (1.0 - u) * c


def _body(x_ref, s_ref, w0_ref, wr0_ref, bg0_ref, bc0_ref,
          w1_ref, wr1_ref, bg1_ref, bc1_ref, wfc_ref, bfc_ref,
          mask_ref, out_ref):
    S = jnp.broadcast_to(s_ref[...], (NP, NN, NN))
    w0 = [w0_ref[k] for k in range(NM)]
    wr0 = [wr0_ref[k] for k in range(NM)]
    w1 = [w1_ref[k] for k in range(NM)]
    wr1 = [wr1_ref[k] for k in range(NM)]
    bg0 = bg0_ref[...]
    bc0 = bc0_ref[...]
    bg1 = bg1_ref[...]
    bc1 = bc1_ref[...]

    def step(t, carry):
        h0, h1, last = carry
        xt = x_ref[t].T.reshape(NP, NN, 2 * INPUT_DIM)
        pair0 = jnp.concatenate([h0, xt], axis=2).astype(jnp.bfloat16)
        h0 = _cell(S, pair0, h0, w0, wr0, bg0, bc0)
        pair1 = jnp.concatenate([h0, h1], axis=2).astype(jnp.bfloat16)
        h1 = _cell(S, pair1, h1, w1, wr1, bg1, bc1)
        last = last + h1 * mask_ref[t].reshape(NP, 1, HL)
        return h0, h1, last

    h0 = jnp.zeros((NP, NN, HL), jnp.float32)
    h1 = jnp.zeros((NP, NN, HL), jnp.float32)
    last = jnp.zeros((NP, NN, HL), jnp.float32)
    h0, h1, last = jax.lax.fori_loop(0, SEQ_LEN, step, (h0, h1, last))

    logits = _dot3(jax.nn.relu(last), wfc_ref[...]) + bfc_ref[...]
    out_ref[...] = jnp.max(logits[:, :NUM_NODES, :], axis=1)


def kernel(input_seq, seq_lengths, supports, Wg0, bg0, Wc0, bc0,
           Wg1, bg1, Wc1, bc1, W_fc, b_fc):
    xseq = jnp.transpose(
        input_seq.reshape(NP, 2, SEQ_LEN, NUM_NODES, INPUT_DIM),
        (2, 1, 4, 0, 3))
    xseq = jnp.pad(xseq, ((0, 0), (0, 0), (0, 0), (0, 0),
                          (0, NN - NUM_NODES))
                   ).reshape(SEQ_LEN, 2 * INPUT_DIM, NP * NN)
    S = jnp.pad(supports[0], ((0, NN - NUM_NODES), (0, NN - NUM_NODES)))

    wg0x, wg0h = _split_xh(Wg0, INPUT_DIM)
    wc0x, wc0h = _split_xh(Wc0, INPUT_DIM)
    wg1x, wg1h = _split_xh(Wg1, RNN_UNITS)
    wc1x, wc1h = _split_xh(Wc1, RNN_UNITS)

    w0, wr0, w1, wr1 = [], [], [], []
    for k in range(NM):
        top = jnp.concatenate(
            [_dup_gate(wg0h[k]), jnp.zeros((HL, HL), jnp.float32)], axis=1)
        bot = jnp.concatenate(
            [_dup_gate(wg0x[k]), _dup_cand(wc0x[k])], axis=1)
        w0.append(jnp.concatenate([top, bot], axis=0))
        wr0.append(_dup_cand(wc0h[k]))
        top = jnp.concatenate(
            [_dup_gate(wg1x[k]), _dup_cand(wc1x[k])], axis=1)
        bot = jnp.concatenate(
            [_dup_gate(wg1h[k]), jnp.zeros((HL, HL), jnp.float32)], axis=1)
        w1.append(jnp.concatenate([top, bot], axis=0))
        wr1.append(_dup_cand(wc1h[k]))
    w0 = jnp.stack(w0)
    wr0 = jnp.stack(wr0)
    w1 = jnp.stack(w1)
    wr1 = jnp.stack(wr1)

    def gate_bias(b):
        return jnp.broadcast_to(b.reshape(2, 1, RNN_UNITS),
                                (2, 2, RNN_UNITS)).reshape(1, GL)

    def cand_bias(b):
        return jnp.broadcast_to(b.reshape(1, RNN_UNITS),
                                (2, RNN_UNITS)).reshape(1, HL)

    eye = jnp.eye(2, dtype=W_fc.dtype)
    wfc2 = jnp.einsum('uc,ab->aubc', W_fc, eye).reshape(HL, 2 * NUM_CLASSES)
    bfc2 = jnp.broadcast_to(b_fc.reshape(1, NUM_CLASSES),
                            (2, NUM_CLASSES)).reshape(1, 2 * NUM_CLASSES)

    idx = jnp.clip(seq_lengths - 1, 0, SEQ_LEN - 1)
    onehot = (jnp.arange(SEQ_LEN)[:, None] == idx[None, :]).astype(jnp.float32)
    mask = jnp.repeat(onehot.reshape(SEQ_LEN, NP, 2, 1), RNN_UNITS,
                      axis=3).reshape(SEQ_LEN, NP, HL)

    bf = jnp.bfloat16
    pooled2 = pl.pallas_call(
        _body,
        out_shape=jax.ShapeDtypeStruct((NP, 2 * NUM_CLASSES), jnp.float32),
    )(xseq, S.astype(bf), w0.astype(bf), wr0.astype(bf),
      gate_bias(bg0), cand_bias(bc0), w1.astype(bf), wr1.astype(bf),
      gate_bias(bg1), cand_bias(bc1), wfc2.astype(bf), bfc2, mask)
    return pooled2.reshape(BATCH, NUM_CLASSES)
